# Initial kernel scaffold; baseline (speedup 1.0000x reference)
import jax
import jax.numpy as jnp
from jax.experimental import pallas as pl


def kernel(t, control):
    raise NotImplementedError("write your pallas kernel here")


# speedup vs baseline: 32.6281x; 32.6281x over previous
import functools

import jax
import jax.numpy as jnp
from jax import lax
from jax.experimental import pallas as pl
from jax.experimental.pallas import tpu as pltpu
from jax.experimental.pallas import tpu_sc as plsc

CH = 256
STEPS = 8192
NQ = 16384
NC = 2
NS = 16
L = 16
NW = NC * NS
QPW = NQ // NW
CHUNK = 128
NCHUNK = QPW // CHUNK


_mesh = plsc.VectorSubcoreMesh(core_axis_name="c", subcore_axis_name="s")


@functools.partial(
    pl.kernel,
    out_type=jax.ShapeDtypeStruct((NQ, CH), jnp.float32),
    mesh=_mesh,
    scratch_types=[
        pltpu.VMEM((QPW,), jnp.float32),
        pltpu.VMEM((QPW,), jnp.int32),
        pltpu.VMEM((QPW,), jnp.int32),
        pltpu.VMEM((QPW,), jnp.float32),
        pltpu.VMEM((CHUNK, CH), jnp.float32),
        pltpu.VMEM((CHUNK, CH), jnp.float32),
        pltpu.VMEM((CHUNK, CH), jnp.float32),
        pltpu.SemaphoreType.DMA,
        pltpu.SemaphoreType.DMA,
    ],
)
def _interp_kernel(t_hbm, control_hbm, out_hbm,
                   t_v, i0_v, i1_v, f_v, r0_v, r1_v, o_v, sem0, sem1):
    wid = lax.axis_index("s") * NC + lax.axis_index("c")
    base = wid * QPW

    pltpu.sync_copy(t_hbm.at[pl.ds(base, QPW)], t_v)

    def precompute(i, carry):
        tv = t_v[pl.ds(i * L, L)]
        pos = tv * jnp.float32(STEPS - 1)
        i0 = pos.astype(jnp.int32)
        i0 = jnp.maximum(jnp.minimum(i0, STEPS - 2), 0)
        fr = pos - i0.astype(jnp.float32)
        i0_v[pl.ds(i * L, L)] = i0
        i1_v[pl.ds(i * L, L)] = i0 + 1
        f_v[pl.ds(i * L, L)] = fr
        return carry

    lax.fori_loop(0, QPW // L, precompute, 0)

    for c in range(NCHUNK):
        cbase = c * CHUNK
        g0 = pltpu.async_copy(
            control_hbm.at[i0_v.at[pl.ds(cbase, CHUNK)]], r0_v, sem0)
        g1 = pltpu.async_copy(
            control_hbm.at[i1_v.at[pl.ds(cbase, CHUNK)]], r1_v, sem1)
        g0.wait()
        g1.wait()

        def blend(qg, carry, cbase=cbase):
            fr16 = f_v[pl.ds(cbase + qg * L, L)]
            for j in range(L):
                q = qg * L + j
                fq = jnp.full((L,), fr16[j])
                for g in range(CH // L):
                    c0 = r0_v[q, pl.ds(g * L, L)]
                    c1 = r1_v[q, pl.ds(g * L, L)]
                    o_v[q, pl.ds(g * L, L)] = c0 + fq * (c1 - c0)
            return carry

        lax.fori_loop(0, CHUNK // L, blend, 0)
        pltpu.sync_copy(o_v, out_hbm.at[pl.ds(base + cbase, CHUNK)])


def kernel(t, control):
    return _interp_kernel(t, control)


# speedup vs baseline: 44.4872x; 1.3635x over previous
import functools

import jax
import jax.numpy as jnp
from jax import lax
from jax.experimental import pallas as pl
from jax.experimental.pallas import tpu as pltpu
from jax.experimental.pallas import tpu_sc as plsc

CH = 256
STEPS = 8192
NQ = 16384
NC = 2
NS = 16
L = 16
NW = NC * NS
QPW = NQ // NW
CHUNK = 64
NCHUNK = QPW // CHUNK
NBUF = 2


_mesh = plsc.VectorSubcoreMesh(core_axis_name="c", subcore_axis_name="s")


@functools.partial(
    pl.kernel,
    out_type=jax.ShapeDtypeStruct((NQ, CH), jnp.float32),
    mesh=_mesh,
    scratch_types=[
        pltpu.VMEM((QPW,), jnp.float32),
        pltpu.VMEM((QPW,), jnp.int32),
        pltpu.VMEM((QPW,), jnp.int32),
        pltpu.VMEM((QPW,), jnp.float32),
        pltpu.VMEM((CHUNK, CH), jnp.float32),
        pltpu.VMEM((CHUNK, CH), jnp.float32),
        pltpu.VMEM((CHUNK, CH), jnp.float32),
        pltpu.VMEM((CHUNK, CH), jnp.float32),
        pltpu.VMEM((CHUNK, CH), jnp.float32),
        pltpu.VMEM((CHUNK, CH), jnp.float32),
        pltpu.SemaphoreType.DMA,
        pltpu.SemaphoreType.DMA,
        pltpu.SemaphoreType.DMA,
        pltpu.SemaphoreType.DMA,
    ],
)
def _interp_kernel(t_hbm, control_hbm, out_hbm,
                   t_v, i0_v, i1_v, f_v,
                   r0a, r0b, r1a, r1b, oa, ob,
                   gsa, gsb, ssa, ssb):
    r0 = (r0a, r0b)
    r1 = (r1a, r1b)
    o = (oa, ob)
    gs = (gsa, gsb)
    ss = (ssa, ssb)

    wid = lax.axis_index("s") * NC + lax.axis_index("c")
    base = wid * QPW

    pltpu.sync_copy(t_hbm.at[pl.ds(base, QPW)], t_v)

    def precompute(i, carry):
        tv = t_v[pl.ds(i * L, L)]
        pos = tv * jnp.float32(STEPS - 1)
        i0 = pos.astype(jnp.int32)
        i0 = jnp.maximum(jnp.minimum(i0, STEPS - 2), 0)
        fr = pos - i0.astype(jnp.float32)
        i0_v[pl.ds(i * L, L)] = i0
        i1_v[pl.ds(i * L, L)] = i0 + 1
        f_v[pl.ds(i * L, L)] = fr
        return carry

    lax.fori_loop(0, QPW // L, precompute, 0)

    def fire_gathers(c, b):
        cb = c * CHUNK
        pltpu.async_copy(
            control_hbm.at[i0_v.at[pl.ds(cb, CHUNK)]], r0[b], gs[b])
        pltpu.async_copy(
            control_hbm.at[i1_v.at[pl.ds(cb, CHUNK)]], r1[b], gs[b])

    def wait_gathers(c, b):
        cb = c * CHUNK
        pltpu.make_async_copy(
            control_hbm.at[i0_v.at[pl.ds(cb, CHUNK)]], r0[b], gs[b]).wait()
        pltpu.make_async_copy(
            control_hbm.at[i1_v.at[pl.ds(cb, CHUNK)]], r1[b], gs[b]).wait()

    fire_gathers(0, 0)
    fire_gathers(1, 1)

    def outer(k, carry):
        for b in range(NBUF):
            c = NBUF * k + b
            cb = c * CHUNK
            wait_gathers(c, b)

            @pl.when(k > 0)
            def _():
                pltpu.make_async_copy(
                    o[b], out_hbm.at[pl.ds(base, CHUNK)], ss[b]).wait()

            def blend(qg, carry2, b=b, cb=cb):
                fr16 = f_v[pl.ds(cb + qg * L, L)]
                for j in range(L):
                    q = qg * L + j
                    fq = jnp.full((L,), fr16[j])
                    for g in range(CH // L):
                        c0 = r0[b][q, pl.ds(g * L, L)]
                        c1 = r1[b][q, pl.ds(g * L, L)]
                        o[b][q, pl.ds(g * L, L)] = c0 + fq * (c1 - c0)
                return carry2

            lax.fori_loop(0, CHUNK // L, blend, 0)

            pltpu.async_copy(o[b], out_hbm.at[pl.ds(base + cb, CHUNK)], ss[b])

            @pl.when(c + NBUF < NCHUNK)
            def _(c=c, b=b):
                fire_gathers(c + NBUF, b)
        return carry

    lax.fori_loop(0, NCHUNK // NBUF, outer, 0)

    pltpu.make_async_copy(o[0], out_hbm.at[pl.ds(base, CHUNK)], ss[0]).wait()
    pltpu.make_async_copy(o[1], out_hbm.at[pl.ds(base, CHUNK)], ss[1]).wait()


def kernel(t, control):
    return _interp_kernel(t, control)
